# baseline (speedup 1.0000x reference)
import functools

import jax
import jax.numpy as jnp
from jax import lax
from jax.experimental import pallas as pl
from jax.experimental.pallas import tpu as pltpu
from jax.experimental.pallas import tpu_sc as plsc

N_STATES = 10000
WINDOW = 256
N_EDGES = 160000
H1, H2, GRU_H = 32, 16, 16

NP = 10240
NC, NS = 2, 16
NW = NC * NS
EB = 128
NBLK = 42
EPW = NBLK * EB
EPAD = NW * EPW
EPC = EPAD // NC
CHK = 2048
NCHK = EPC // CHK
NPW = NP // NS

_mesh = plsc.VectorSubcoreMesh(core_axis_name="c", subcore_axis_name="s")
_sc_params = pltpu.CompilerParams(needs_layout_passes=False,
                                  use_tc_tiling_on_sc=False)


def _wid():
    return lax.axis_index("c") * NS + lax.axis_index("s")


@functools.partial(
    pl.kernel,
    out_type=jax.ShapeDtypeStruct((NC, NP), jnp.float32),
    mesh=_mesh,
    scratch_types=[
        pltpu.VMEM((NBLK, EB), jnp.int32),
        pltpu.VMEM((NBLK, EB), jnp.float32),
        pltpu.VMEM((NPW,), jnp.float32),
        pltpu.VMEM_SHARED((NP,), jnp.float32),
    ],
    compiler_params=_sc_params,
)
def _sc_deg(col_hbm, ew_hbm, out_hbm, colv, ewv, zbuf, acc_sh):
    cid = lax.axis_index("c")
    sid = lax.axis_index("s")
    wid = _wid()

    def zb(i, _):
        zbuf[pl.ds(i * 16, 16)] = jnp.zeros((16,), jnp.float32)
        return _

    lax.fori_loop(0, NPW // 16, zb, None, unroll=8)
    pltpu.sync_copy(zbuf, acc_sh.at[pl.ds(sid * NPW, NPW)])
    plsc.subcore_barrier()

    pltpu.sync_copy(col_hbm.at[wid], colv)
    pltpu.sync_copy(ew_hbm.at[wid], ewv)

    def body(j, _):
        pltpu.sync_copy(ewv.at[j], acc_sh.at[colv.at[j]], add=True)
        return _

    lax.fori_loop(0, NBLK, body, None)
    plsc.subcore_barrier()
    pltpu.sync_copy(acc_sh.at[pl.ds(sid * NPW, NPW)],
                    out_hbm.at[cid, pl.ds(sid * NPW, NPW)])


@functools.partial(
    pl.kernel,
    out_type=jax.ShapeDtypeStruct((NW, EPW), jnp.float32),
    mesh=_mesh,
    scratch_types=[
        pltpu.VMEM((NP,), jnp.float32),
        pltpu.VMEM((EPW,), jnp.int32),
        pltpu.VMEM((EPW,), jnp.int32),
        pltpu.VMEM((EPW,), jnp.float32),
    ],
    compiler_params=_sc_params,
)
def _sc_norm(dinv_hbm, row_hbm, col_hbm, ew_hbm, out_hbm,
             dinv, rowv, colv, ewv):
    wid = _wid()
    pltpu.sync_copy(dinv_hbm, dinv)
    pltpu.sync_copy(row_hbm.at[wid], rowv)
    pltpu.sync_copy(col_hbm.at[wid], colv)
    pltpu.sync_copy(ew_hbm.at[wid], ewv)

    def ebody(i, _):
        sl = pl.ds(i * 16, 16)
        dr = plsc.load_gather(dinv, [rowv[sl]])
        dc = plsc.load_gather(dinv, [colv[sl]])
        ewv[sl] = dr * ewv[sl] * dc
        return _

    lax.fori_loop(0, EPW // 16, ebody, None, unroll=8)
    pltpu.sync_copy(ewv, out_hbm.at[wid])


def _make_sc_agg(F):
    FB = F // NS

    @functools.partial(
        pl.kernel,
        out_type=jax.ShapeDtypeStruct((NC, F, NP), jnp.float32),
        mesh=_mesh,
        scratch_types=[
            [pltpu.VMEM((NP,), jnp.float32) for _ in range(FB)],
            [pltpu.VMEM((NP,), jnp.float32) for _ in range(FB)],
            [pltpu.VMEM((CHK,), jnp.int32) for _ in range(2)],
            [pltpu.VMEM((CHK,), jnp.int32) for _ in range(2)],
            [pltpu.VMEM((CHK,), jnp.float32) for _ in range(2)],
            [pltpu.SemaphoreType.DMA for _ in range(2)],
        ],
        compiler_params=_sc_params,
    )
    def _sc_agg(h_hbm, row_hbm, col_hbm, norm_hbm, out_hbm,
                hcols, accs, rows, cols, nrms, sems):
        cid = lax.axis_index("c")
        sid = lax.axis_index("s")

        for k in range(FB):
            pltpu.async_copy(h_hbm.at[sid * FB + k], hcols[k], sems[0])

        def za(i, _):
            sl = pl.ds(i * 16, 16)
            for k in range(FB):
                accs[k][sl] = jnp.zeros((16,), jnp.float32)
            return _

        lax.fori_loop(0, NP // 16, za, None, unroll=8)
        for k in range(FB):
            pltpu.make_async_copy(h_hbm.at[sid * FB + k], hcols[k],
                                  sems[0]).wait()

        def start_chunk(ci, b):
            base = cid * EPC + ci * CHK
            pltpu.async_copy(row_hbm.at[pl.ds(base, CHK)], rows[b], sems[b])
            pltpu.async_copy(col_hbm.at[pl.ds(base, CHK)], cols[b], sems[b])
            pltpu.async_copy(norm_hbm.at[pl.ds(base, CHK)], nrms[b], sems[b])

        def wait_chunk(ci, b):
            base = cid * EPC + ci * CHK
            pltpu.make_async_copy(row_hbm.at[pl.ds(base, CHK)], rows[b],
                                  sems[b]).wait()
            pltpu.make_async_copy(col_hbm.at[pl.ds(base, CHK)], cols[b],
                                  sems[b]).wait()
            pltpu.make_async_copy(norm_hbm.at[pl.ds(base, CHK)], nrms[b],
                                  sems[b]).wait()

        def process(b):
            def ibody(i, _):
                sl = pl.ds(i * 16, 16)
                r16 = rows[b][sl]
                c16 = cols[b][sl]
                n16 = nrms[b][sl]
                for k in range(FB):
                    v = plsc.load_gather(hcols[k], [r16]) * n16
                    plsc.addupdate_scatter(accs[k], [c16], v)
                return _

            lax.fori_loop(0, CHK // 16, ibody, None, unroll=8)

        start_chunk(0, 0)

        def body(ck, _):
            c0, c1 = 2 * ck, 2 * ck + 1
            wait_chunk(c0, 0)
            start_chunk(c1, 1)
            process(0)
            wait_chunk(c1, 1)

            @pl.when(ck < NCHK // 2 - 1)
            def _n():
                start_chunk(c0 + 2, 0)

            process(1)
            return _

        lax.fori_loop(0, NCHK // 2, body, None)
        for k in range(FB):
            pltpu.sync_copy(accs[k], out_hbm.at[cid, sid * FB + k])

    return _sc_agg


_sc_agg1 = _make_sc_agg(H1)
_sc_agg2 = _make_sc_agg(H2)


def _tc_h1_body(seq_ref, w1_ref, degp_ref, out_ref, dinv_ref):
    out_ref[...] = lax.dot_general(
        w1_ref[...], seq_ref[...], (((1,), (1,)), ((), ())),
        preferred_element_type=jnp.float32)
    dinv_ref[...] = lax.rsqrt(degp_ref[0] + degp_ref[1])


def _tc_mid_body(p_ref, b1_ref, w2_ref, out_ref):
    x1 = jax.nn.relu(p_ref[0] + p_ref[1] + b1_ref[...])
    out_ref[...] = lax.dot_general(
        w2_ref[...], x1, (((1,), (0,)), ((), ())),
        preferred_element_type=jnp.float32)


def _tc_fin_body(q_ref, b2_ref, wr_ref, wz_ref, wn_ref,
                 br_ref, bz_ref, bni_ref, bnh_ref, wf_ref, bf_ref, out_ref):
    x2 = jax.nn.relu(q_ref[0] + q_ref[1] + b2_ref[...])

    def mm(w, x):
        return lax.dot_general(w[...], x, (((1,), (0,)), ((), ())),
                               preferred_element_type=jnp.float32)

    r = jax.nn.sigmoid(mm(wr_ref, x2) + br_ref[...])
    z = jax.nn.sigmoid(mm(wz_ref, x2) + bz_ref[...])
    n = jnp.tanh(mm(wn_ref, x2) + bni_ref[...] + r * bnh_ref[...])
    hn = (1.0 - z) * n
    out_ref[...] = mm(wf_ref, hn) + bf_ref[0]


def _tc_call(body, out_shape, *args, in_specs=None):
    kw = {} if in_specs is None else {"in_specs": in_specs}
    return pl.pallas_call(body, out_shape=out_shape, **kw)(*args)


def kernel(sequences, edge_weight, W1, b1, W2, b2, W_ih, W_hh, b_ih, b_hh,
           Wf, bf, state_indices, edge_index):
    f32, i32 = jnp.float32, jnp.int32

    seq_pad = jnp.zeros((NP, WINDOW), f32).at[:N_STATES].set(sequences)
    nfill = EPAD - N_EDGES - NP
    loops = jnp.arange(NP, dtype=i32)
    fill_i = jnp.zeros((nfill,), i32)
    row = jnp.concatenate([edge_index[0], loops, fill_i])
    col = jnp.concatenate([edge_index[1], loops, fill_i])
    ew = jnp.concatenate([edge_weight, jnp.ones((NP,), f32),
                          jnp.zeros((nfill,), f32)])
    row2 = row.reshape(NW, NBLK, EB)
    col2 = col.reshape(NW, NBLK, EB)
    ew2 = ew.reshape(NW, NBLK, EB)
    rowf = row.reshape(NW, EPW)
    colf = col.reshape(NW, EPW)
    ewf = ew.reshape(NW, EPW)

    degp = _sc_deg(col2, ew2)
    h1t, dinv = _tc_call(_tc_h1_body,
                         (jax.ShapeDtypeStruct((H1, NP), f32),
                          jax.ShapeDtypeStruct((NP,), f32)),
                         seq_pad, W1, degp)
    norm = _sc_norm(dinv, rowf, colf, ewf).reshape(EPAD)

    p1 = _sc_agg1(h1t, row, col, norm)
    h2t = _tc_call(_tc_mid_body, jax.ShapeDtypeStruct((H2, NP), f32),
                   p1, b1.reshape(H1, 1), W2)
    p2 = _sc_agg2(h2t, row, col, norm)

    Wr, Wz, Wn = W_ih[:GRU_H], W_ih[GRU_H:2 * GRU_H], W_ih[2 * GRU_H:]
    br = (b_ih[:GRU_H] + b_hh[:GRU_H]).reshape(GRU_H, 1)
    bz = (b_ih[GRU_H:2 * GRU_H] + b_hh[GRU_H:2 * GRU_H]).reshape(GRU_H, 1)
    bni = b_ih[2 * GRU_H:].reshape(GRU_H, 1)
    bnh = b_hh[2 * GRU_H:].reshape(GRU_H, 1)
    Wf8 = jnp.zeros((8, GRU_H), f32).at[:1].set(Wf)
    out = _tc_call(_tc_fin_body, jax.ShapeDtypeStruct((8, NP), f32),
                   p2, b2.reshape(H2, 1), Wr, Wz, Wn, br, bz, bni, bnh,
                   Wf8, bf,
                   in_specs=[pl.BlockSpec()] * 10
                   + [pl.BlockSpec(memory_space=pltpu.SMEM)])
    return out[0, :N_STATES].reshape(N_STATES, 1)


# speedup vs baseline: 1.4188x; 1.4188x over previous
import functools

import jax
import jax.numpy as jnp
from jax import lax
from jax.experimental import pallas as pl
from jax.experimental.pallas import tpu as pltpu
from jax.experimental.pallas import tpu_sc as plsc

N_STATES = 10000
WINDOW = 256
N_EDGES = 160000
H1, H2, GRU_H = 32, 16, 16

NP = 10240
NC, NS = 2, 16
NW = NC * NS
EB = 128
NBLK = 42
EPW = NBLK * EB
EPAD = NW * EPW
EPC = EPAD // NC
CHK = 2048
NCHK = EPC // CHK
NPW = NP // NS

_mesh = plsc.VectorSubcoreMesh(core_axis_name="c", subcore_axis_name="s")
_sc_params = pltpu.CompilerParams(needs_layout_passes=False,
                                  use_tc_tiling_on_sc=False)


def _wid():
    return lax.axis_index("c") * NS + lax.axis_index("s")


@functools.partial(
    pl.kernel,
    out_type=jax.ShapeDtypeStruct((NC, NP), jnp.float32),
    mesh=_mesh,
    scratch_types=[
        pltpu.VMEM((NBLK, EB), jnp.int32),
        pltpu.VMEM((NBLK, EB), jnp.float32),
        pltpu.VMEM((NPW,), jnp.float32),
        pltpu.VMEM_SHARED((NP,), jnp.float32),
    ],
    compiler_params=_sc_params,
)
def _sc_deg(col_hbm, ew_hbm, out_hbm, colv, ewv, zbuf, acc_sh):
    cid = lax.axis_index("c")
    sid = lax.axis_index("s")
    wid = _wid()

    def zb(i, _):
        zbuf[pl.ds(i * 16, 16)] = jnp.zeros((16,), jnp.float32)
        return _

    lax.fori_loop(0, NPW // 16, zb, None, unroll=8)
    pltpu.sync_copy(zbuf, acc_sh.at[pl.ds(sid * NPW, NPW)])
    plsc.subcore_barrier()

    pltpu.sync_copy(col_hbm.at[wid], colv)
    pltpu.sync_copy(ew_hbm.at[wid], ewv)

    def body(j, _):
        pltpu.sync_copy(ewv.at[j], acc_sh.at[colv.at[j]], add=True)
        return _

    lax.fori_loop(0, NBLK, body, None)
    plsc.subcore_barrier()
    pltpu.sync_copy(acc_sh.at[pl.ds(sid * NPW, NPW)],
                    out_hbm.at[cid, pl.ds(sid * NPW, NPW)])


@functools.partial(
    pl.kernel,
    out_type=jax.ShapeDtypeStruct((NW, EPW), jnp.float32),
    mesh=_mesh,
    scratch_types=[
        pltpu.VMEM((NP,), jnp.float32),
        pltpu.VMEM((EPW,), jnp.int32),
        pltpu.VMEM((EPW,), jnp.int32),
        pltpu.VMEM((EPW,), jnp.float32),
    ],
    compiler_params=_sc_params,
)
def _sc_norm(dinv_hbm, row_hbm, col_hbm, ew_hbm, out_hbm,
             dinv, rowv, colv, ewv):
    wid = _wid()
    pltpu.sync_copy(dinv_hbm, dinv)
    pltpu.sync_copy(row_hbm.at[wid], rowv)
    pltpu.sync_copy(col_hbm.at[wid], colv)
    pltpu.sync_copy(ew_hbm.at[wid], ewv)

    @plsc.parallel_loop(0, EPW // 16, unroll=8)
    def ebody(i):
        sl = pl.ds(i * 16, 16)
        dr = plsc.load_gather(dinv, [rowv[sl]])
        dc = plsc.load_gather(dinv, [colv[sl]])
        ewv[sl] = dr * ewv[sl] * dc

    pltpu.sync_copy(ewv, out_hbm.at[wid])


def _make_sc_agg(F):
    FB = F // NS

    @functools.partial(
        pl.kernel,
        out_type=jax.ShapeDtypeStruct((NC, F, NP), jnp.float32),
        mesh=_mesh,
        scratch_types=[
            [pltpu.VMEM((NP,), jnp.float32) for _ in range(FB)],
            [pltpu.VMEM((NP,), jnp.float32) for _ in range(FB)],
            [pltpu.VMEM((CHK,), jnp.int32) for _ in range(2)],
            [pltpu.VMEM((CHK,), jnp.int32) for _ in range(2)],
            [pltpu.VMEM((CHK,), jnp.float32) for _ in range(2)],
            [pltpu.SemaphoreType.DMA for _ in range(2)],
        ],
        compiler_params=_sc_params,
    )
    def _sc_agg(h_hbm, row_hbm, col_hbm, norm_hbm, out_hbm,
                hcols, accs, rows, cols, nrms, sems):
        cid = lax.axis_index("c")
        sid = lax.axis_index("s")

        for k in range(FB):
            pltpu.async_copy(h_hbm.at[sid * FB + k], hcols[k], sems[0])

        def za(i, _):
            sl = pl.ds(i * 16, 16)
            for k in range(FB):
                accs[k][sl] = jnp.zeros((16,), jnp.float32)
            return _

        lax.fori_loop(0, NP // 16, za, None, unroll=8)
        for k in range(FB):
            pltpu.make_async_copy(h_hbm.at[sid * FB + k], hcols[k],
                                  sems[0]).wait()

        def start_chunk(ci, b):
            base = cid * EPC + ci * CHK
            pltpu.async_copy(row_hbm.at[pl.ds(base, CHK)], rows[b], sems[b])
            pltpu.async_copy(col_hbm.at[pl.ds(base, CHK)], cols[b], sems[b])
            pltpu.async_copy(norm_hbm.at[pl.ds(base, CHK)], nrms[b], sems[b])

        def wait_chunk(ci, b):
            base = cid * EPC + ci * CHK
            pltpu.make_async_copy(row_hbm.at[pl.ds(base, CHK)], rows[b],
                                  sems[b]).wait()
            pltpu.make_async_copy(col_hbm.at[pl.ds(base, CHK)], cols[b],
                                  sems[b]).wait()
            pltpu.make_async_copy(norm_hbm.at[pl.ds(base, CHK)], nrms[b],
                                  sems[b]).wait()

        def process(b):
            @plsc.parallel_loop(0, CHK // 16, unroll=8)
            def ibody(i):
                sl = pl.ds(i * 16, 16)
                r16 = rows[b][sl]
                c16 = cols[b][sl]
                n16 = nrms[b][sl]
                for k in range(FB):
                    v = plsc.load_gather(hcols[k], [r16]) * n16
                    plsc.addupdate_scatter(accs[k], [c16], v)

        start_chunk(0, 0)

        def body(ck, _):
            c0, c1 = 2 * ck, 2 * ck + 1
            wait_chunk(c0, 0)
            start_chunk(c1, 1)
            process(0)
            wait_chunk(c1, 1)

            @pl.when(ck < NCHK // 2 - 1)
            def _n():
                start_chunk(c0 + 2, 0)

            process(1)
            return _

        lax.fori_loop(0, NCHK // 2, body, None)
        for k in range(FB):
            pltpu.sync_copy(accs[k], out_hbm.at[cid, sid * FB + k])

    return _sc_agg


_sc_agg1 = _make_sc_agg(H1)
_sc_agg2 = _make_sc_agg(H2)


def _tc_h1_body(seq_ref, w1_ref, degp_ref, out_ref, dinv_ref):
    out_ref[...] = lax.dot_general(
        w1_ref[...], seq_ref[...], (((1,), (1,)), ((), ())),
        preferred_element_type=jnp.float32)
    dinv_ref[...] = lax.rsqrt(degp_ref[0] + degp_ref[1])


def _tc_mid_body(p_ref, b1_ref, w2_ref, out_ref):
    x1 = jax.nn.relu(p_ref[0] + p_ref[1] + b1_ref[...])
    out_ref[...] = lax.dot_general(
        w2_ref[...], x1, (((1,), (0,)), ((), ())),
        preferred_element_type=jnp.float32)


def _tc_fin_body(q_ref, b2_ref, wr_ref, wz_ref, wn_ref,
                 br_ref, bz_ref, bni_ref, bnh_ref, wf_ref, bf_ref, out_ref):
    x2 = jax.nn.relu(q_ref[0] + q_ref[1] + b2_ref[...])

    def mm(w, x):
        return lax.dot_general(w[...], x, (((1,), (0,)), ((), ())),
                               preferred_element_type=jnp.float32)

    r = jax.nn.sigmoid(mm(wr_ref, x2) + br_ref[...])
    z = jax.nn.sigmoid(mm(wz_ref, x2) + bz_ref[...])
    n = jnp.tanh(mm(wn_ref, x2) + bni_ref[...] + r * bnh_ref[...])
    hn = (1.0 - z) * n
    out_ref[...] = mm(wf_ref, hn) + bf_ref[0]


def _tc_call(body, out_shape, *args, in_specs=None):
    kw = {} if in_specs is None else {"in_specs": in_specs}
    return pl.pallas_call(body, out_shape=out_shape, **kw)(*args)


def kernel(sequences, edge_weight, W1, b1, W2, b2, W_ih, W_hh, b_ih, b_hh,
           Wf, bf, state_indices, edge_index):
    f32, i32 = jnp.float32, jnp.int32

    seq_pad = jnp.zeros((NP, WINDOW), f32).at[:N_STATES].set(sequences)
    nfill = EPAD - N_EDGES - NP
    loops = jnp.arange(NP, dtype=i32)
    fill_i = jnp.zeros((nfill,), i32)
    row = jnp.concatenate([edge_index[0], loops, fill_i])
    col = jnp.concatenate([edge_index[1], loops, fill_i])
    ew = jnp.concatenate([edge_weight, jnp.ones((NP,), f32),
                          jnp.zeros((nfill,), f32)])
    row2 = row.reshape(NW, NBLK, EB)
    col2 = col.reshape(NW, NBLK, EB)
    ew2 = ew.reshape(NW, NBLK, EB)
    rowf = row.reshape(NW, EPW)
    colf = col.reshape(NW, EPW)
    ewf = ew.reshape(NW, EPW)

    degp = _sc_deg(col2, ew2)
    h1t, dinv = _tc_call(_tc_h1_body,
                         (jax.ShapeDtypeStruct((H1, NP), f32),
                          jax.ShapeDtypeStruct((NP,), f32)),
                         seq_pad, W1, degp)
    norm = _sc_norm(dinv, rowf, colf, ewf).reshape(EPAD)

    p1 = _sc_agg1(h1t, row, col, norm)
    h2t = _tc_call(_tc_mid_body, jax.ShapeDtypeStruct((H2, NP), f32),
                   p1, b1.reshape(H1, 1), W2)
    p2 = _sc_agg2(h2t, row, col, norm)

    Wr, Wz, Wn = W_ih[:GRU_H], W_ih[GRU_H:2 * GRU_H], W_ih[2 * GRU_H:]
    br = (b_ih[:GRU_H] + b_hh[:GRU_H]).reshape(GRU_H, 1)
    bz = (b_ih[GRU_H:2 * GRU_H] + b_hh[GRU_H:2 * GRU_H]).reshape(GRU_H, 1)
    bni = b_ih[2 * GRU_H:].reshape(GRU_H, 1)
    bnh = b_hh[2 * GRU_H:].reshape(GRU_H, 1)
    Wf8 = jnp.zeros((8, GRU_H), f32).at[:1].set(Wf)
    out = _tc_call(_tc_fin_body, jax.ShapeDtypeStruct((8, NP), f32),
                   p2, b2.reshape(H2, 1), Wr, Wz, Wn, br, bz, bni, bnh,
                   Wf8, bf,
                   in_specs=[pl.BlockSpec()] * 10
                   + [pl.BlockSpec(memory_space=pltpu.SMEM)])
    return out[0, :N_STATES].reshape(N_STATES, 1)


# speedup vs baseline: 1.7599x; 1.2404x over previous
import functools

import jax
import jax.numpy as jnp
from jax import lax
from jax.experimental import pallas as pl
from jax.experimental.pallas import tpu as pltpu
from jax.experimental.pallas import tpu_sc as plsc

N_STATES = 10000
WINDOW = 256
N_EDGES = 160000
H1, H2, GRU_H = 32, 16, 16

NP = 10240
NC, NS = 2, 16
NW = NC * NS
EB = 128
NBLK = 42
EPW = NBLK * EB
EPAD = NW * EPW
EPC = EPAD // NC
CHK = 3072
NCHK = EPC // CHK
NPW = NP // NS

_mesh = plsc.VectorSubcoreMesh(core_axis_name="c", subcore_axis_name="s")
_sc_params = pltpu.CompilerParams(needs_layout_passes=False,
                                  use_tc_tiling_on_sc=False)


def _wid():
    return lax.axis_index("c") * NS + lax.axis_index("s")


@functools.partial(
    pl.kernel,
    out_type=jax.ShapeDtypeStruct((NC, NP), jnp.float32),
    mesh=_mesh,
    scratch_types=[
        pltpu.VMEM((NBLK, EB), jnp.int32),
        pltpu.VMEM((NBLK, EB), jnp.float32),
        pltpu.VMEM((NPW,), jnp.float32),
        pltpu.VMEM_SHARED((NP,), jnp.float32),
    ],
    compiler_params=_sc_params,
)
def _sc_deg(col_hbm, ew_hbm, out_hbm, colv, ewv, zbuf, acc_sh):
    cid = lax.axis_index("c")
    sid = lax.axis_index("s")
    wid = _wid()

    def zb(i, _):
        zbuf[pl.ds(i * 16, 16)] = jnp.zeros((16,), jnp.float32)
        return _

    lax.fori_loop(0, NPW // 16, zb, None, unroll=8)
    pltpu.sync_copy(zbuf, acc_sh.at[pl.ds(sid * NPW, NPW)])
    plsc.subcore_barrier()

    pltpu.sync_copy(col_hbm.at[wid], colv)
    pltpu.sync_copy(ew_hbm.at[wid], ewv)

    def body(j, _):
        pltpu.sync_copy(ewv.at[j], acc_sh.at[colv.at[j]], add=True)
        return _

    lax.fori_loop(0, NBLK, body, None)
    plsc.subcore_barrier()
    pltpu.sync_copy(acc_sh.at[pl.ds(sid * NPW, NPW)],
                    out_hbm.at[cid, pl.ds(sid * NPW, NPW)])


def _make_sc_agg(F):
    FB = F // NS

    @functools.partial(
        pl.kernel,
        out_type=jax.ShapeDtypeStruct((NC, F, NP), jnp.float32),
        mesh=_mesh,
        scratch_types=[
            [pltpu.VMEM((NP,), jnp.float32) for _ in range(FB)],
            [pltpu.VMEM((NP,), jnp.float32) for _ in range(FB)],
            [pltpu.VMEM((CHK,), jnp.int32) for _ in range(2)],
            [pltpu.VMEM((CHK,), jnp.float32) for _ in range(2)],
            [pltpu.SemaphoreType.DMA for _ in range(2)],
        ],
        compiler_params=_sc_params,
    )
    def _sc_agg(h_hbm, rc_hbm, ew_hbm, out_hbm, hcols, accs, rcs, ews, sems):
        cid = lax.axis_index("c")
        sid = lax.axis_index("s")

        for k in range(FB):
            pltpu.async_copy(h_hbm.at[sid * FB + k], hcols[k], sems[0])

        def za(i, _):
            sl = pl.ds(i * 16, 16)
            for k in range(FB):
                accs[k][sl] = jnp.zeros((16,), jnp.float32)
            return _

        lax.fori_loop(0, NP // 16, za, None, unroll=8)
        for k in range(FB):
            pltpu.make_async_copy(h_hbm.at[sid * FB + k], hcols[k],
                                  sems[0]).wait()

        def start_chunk(ci, b):
            base = cid * EPC + ci * CHK
            pltpu.async_copy(rc_hbm.at[pl.ds(base, CHK)], rcs[b], sems[b])
            pltpu.async_copy(ew_hbm.at[pl.ds(base, CHK)], ews[b], sems[b])

        def wait_chunk(ci, b):
            base = cid * EPC + ci * CHK
            pltpu.make_async_copy(rc_hbm.at[pl.ds(base, CHK)], rcs[b],
                                  sems[b]).wait()
            pltpu.make_async_copy(ew_hbm.at[pl.ds(base, CHK)], ews[b],
                                  sems[b]).wait()

        def process(b):
            @plsc.parallel_loop(0, CHK // 16, unroll=8)
            def ibody(i):
                sl = pl.ds(i * 16, 16)
                e16 = rcs[b][sl]
                r16 = e16 & 16383
                c16 = e16 >> 14
                w16 = ews[b][sl]
                for k in range(FB):
                    v = plsc.load_gather(hcols[k], [r16]) * w16
                    plsc.addupdate_scatter(accs[k], [c16], v)

        start_chunk(0, 0)

        def body(ck, _):
            c0, c1 = 2 * ck, 2 * ck + 1
            wait_chunk(c0, 0)
            start_chunk(c1, 1)
            process(0)
            wait_chunk(c1, 1)

            @pl.when(ck < NCHK // 2 - 1)
            def _n():
                start_chunk(c0 + 2, 0)

            process(1)
            return _

        lax.fori_loop(0, NCHK // 2, body, None)
        for k in range(FB):
            pltpu.sync_copy(accs[k], out_hbm.at[cid, sid * FB + k])

    return _sc_agg


_sc_agg1 = _make_sc_agg(H1)
_sc_agg2 = _make_sc_agg(H2)


def _tc_h1_body(seq_ref, w1_ref, degp_ref, out_ref):
    dinv = lax.rsqrt(degp_ref[0] + degp_ref[1])
    out_ref[...] = lax.dot_general(
        w1_ref[...], seq_ref[...], (((1,), (1,)), ((), ())),
        preferred_element_type=jnp.float32) * dinv[None, :]


def _tc_mid_body(p_ref, degp_ref, b1_ref, w2_ref, out_ref):
    dinv = lax.rsqrt(degp_ref[0] + degp_ref[1])
    x1 = jax.nn.relu((p_ref[0] + p_ref[1]) * dinv[None, :] + b1_ref[...])
    out_ref[...] = lax.dot_general(
        w2_ref[...], x1, (((1,), (0,)), ((), ())),
        preferred_element_type=jnp.float32) * dinv[None, :]


def _tc_fin_body(q_ref, degp_ref, b2_ref, wr_ref, wz_ref, wn_ref,
                 br_ref, bz_ref, bni_ref, bnh_ref, wf_ref, bf_ref, out_ref):
    dinv = lax.rsqrt(degp_ref[0] + degp_ref[1])
    x2 = jax.nn.relu((q_ref[0] + q_ref[1]) * dinv[None, :] + b2_ref[...])

    def mm(w, x):
        return lax.dot_general(w[...], x, (((1,), (0,)), ((), ())),
                               preferred_element_type=jnp.float32)

    r = jax.nn.sigmoid(mm(wr_ref, x2) + br_ref[...])
    z = jax.nn.sigmoid(mm(wz_ref, x2) + bz_ref[...])
    n = jnp.tanh(mm(wn_ref, x2) + bni_ref[...] + r * bnh_ref[...])
    hn = (1.0 - z) * n
    out_ref[...] = mm(wf_ref, hn) + bf_ref[0]


def _tc_call(body, out_shape, *args, in_specs=None):
    kw = {} if in_specs is None else {"in_specs": in_specs}
    return pl.pallas_call(body, out_shape=out_shape, **kw)(*args)


def kernel(sequences, edge_weight, W1, b1, W2, b2, W_ih, W_hh, b_ih, b_hh,
           Wf, bf, state_indices, edge_index):
    f32, i32 = jnp.float32, jnp.int32

    seq_pad = jnp.zeros((NP, WINDOW), f32).at[:N_STATES].set(sequences)
    nfill = EPAD - N_EDGES - NP
    loops = jnp.arange(NP, dtype=i32)
    fill_i = jnp.zeros((nfill,), i32)
    row = jnp.concatenate([edge_index[0], loops, fill_i])
    col = jnp.concatenate([edge_index[1], loops, fill_i])
    ew = jnp.concatenate([edge_weight, jnp.ones((NP,), f32),
                          jnp.zeros((nfill,), f32)])
    rc = row | (col << 14)
    col2 = col.reshape(NW, NBLK, EB)
    ew2 = ew.reshape(NW, NBLK, EB)

    degp = _sc_deg(col2, ew2)
    h1t = _tc_call(_tc_h1_body, jax.ShapeDtypeStruct((H1, NP), f32),
                   seq_pad, W1, degp)
    p1 = _sc_agg1(h1t, rc, ew)
    h2t = _tc_call(_tc_mid_body, jax.ShapeDtypeStruct((H2, NP), f32),
                   p1, degp, b1.reshape(H1, 1), W2)
    p2 = _sc_agg2(h2t, rc, ew)

    Wr, Wz, Wn = W_ih[:GRU_H], W_ih[GRU_H:2 * GRU_H], W_ih[2 * GRU_H:]
    br = (b_ih[:GRU_H] + b_hh[:GRU_H]).reshape(GRU_H, 1)
    bz = (b_ih[GRU_H:2 * GRU_H] + b_hh[GRU_H:2 * GRU_H]).reshape(GRU_H, 1)
    bni = b_ih[2 * GRU_H:].reshape(GRU_H, 1)
    bnh = b_hh[2 * GRU_H:].reshape(GRU_H, 1)
    Wf8 = jnp.zeros((8, GRU_H), f32).at[:1].set(Wf)
    out = _tc_call(_tc_fin_body, jax.ShapeDtypeStruct((8, NP), f32),
                   p2, degp, b2.reshape(H2, 1), Wr, Wz, Wn, br, bz, bni, bnh,
                   Wf8, bf,
                   in_specs=[pl.BlockSpec()] * 11
                   + [pl.BlockSpec(memory_space=pltpu.SMEM)])
    return out[0, :N_STATES].reshape(N_STATES, 1)


# speedup vs baseline: 1.8405x; 1.0458x over previous
import functools

import jax
import jax.numpy as jnp
from jax import lax
from jax.experimental import pallas as pl
from jax.experimental.pallas import tpu as pltpu
from jax.experimental.pallas import tpu_sc as plsc

N_STATES = 10000
WINDOW = 256
N_EDGES = 160000
H1, H2, GRU_H = 32, 16, 16

NP = 10240
NC, NS = 2, 16
NW = NC * NS
EB = 128
NBLK = 42
EPW = NBLK * EB
EPAD = NW * EPW
EPC = EPAD // NC
CHK = 3072
NCHK = EPC // CHK
NPW = NP // NS

_mesh = plsc.VectorSubcoreMesh(core_axis_name="c", subcore_axis_name="s")
_sc_params = pltpu.CompilerParams(needs_layout_passes=False,
                                  use_tc_tiling_on_sc=False)


def _wid():
    return lax.axis_index("c") * NS + lax.axis_index("s")


@functools.partial(
    pl.kernel,
    out_type=jax.ShapeDtypeStruct((NC, NP), jnp.float32),
    mesh=_mesh,
    scratch_types=[
        pltpu.VMEM((NBLK, EB), jnp.int32),
        pltpu.VMEM((NBLK, EB), jnp.float32),
        pltpu.VMEM((NPW,), jnp.float32),
        pltpu.VMEM_SHARED((NP,), jnp.float32),
    ],
    compiler_params=_sc_params,
)
def _sc_deg(col_hbm, ew_hbm, out_hbm, colv, ewv, zbuf, acc_sh):
    cid = lax.axis_index("c")
    sid = lax.axis_index("s")
    wid = _wid()

    def zb(i, _):
        zbuf[pl.ds(i * 16, 16)] = jnp.zeros((16,), jnp.float32)
        return _

    lax.fori_loop(0, NPW // 16, zb, None, unroll=8)
    pltpu.sync_copy(zbuf, acc_sh.at[pl.ds(sid * NPW, NPW)])
    plsc.subcore_barrier()

    pltpu.sync_copy(col_hbm.at[wid], colv)
    pltpu.sync_copy(ew_hbm.at[wid], ewv)

    def body(j, _):
        pltpu.sync_copy(ewv.at[j], acc_sh.at[colv.at[j]], add=True)
        return _

    lax.fori_loop(0, NBLK, body, None)
    plsc.subcore_barrier()
    pltpu.sync_copy(acc_sh.at[pl.ds(sid * NPW, NPW)],
                    out_hbm.at[cid, pl.ds(sid * NPW, NPW)])


def _make_sc_agg(F):
    FB = F // 8
    EH = 2
    EPH = EPC // EH
    NCHKH = EPH // CHK

    @functools.partial(
        pl.kernel,
        out_type=jax.ShapeDtypeStruct((NC, EH, F, NP), jnp.float32),
        mesh=_mesh,
        scratch_types=[
            [pltpu.VMEM((NP,), jnp.float32) for _ in range(FB)],
            [pltpu.VMEM((NP,), jnp.float32) for _ in range(FB)],
            [pltpu.VMEM((CHK,), jnp.int32) for _ in range(2)],
            [pltpu.VMEM((CHK,), jnp.float32) for _ in range(2)],
            [pltpu.SemaphoreType.DMA for _ in range(2)],
        ],
        compiler_params=_sc_params,
    )
    def _sc_agg(h_hbm, rc_hbm, ew_hbm, out_hbm, hcols, accs, rcs, ews, sems):
        cid = lax.axis_index("c")
        sid = lax.axis_index("s")
        fg = sid // EH
        eh = sid % EH

        for k in range(FB):
            pltpu.async_copy(h_hbm.at[fg * FB + k], hcols[k], sems[0])

        def za(i, _):
            sl = pl.ds(i * 16, 16)
            for k in range(FB):
                accs[k][sl] = jnp.zeros((16,), jnp.float32)
            return _

        lax.fori_loop(0, NP // 16, za, None, unroll=8)
        for k in range(FB):
            pltpu.make_async_copy(h_hbm.at[fg * FB + k], hcols[k],
                                  sems[0]).wait()

        def start_chunk(ci, b):
            base = cid * EPC + eh * EPH + ci * CHK
            pltpu.async_copy(rc_hbm.at[pl.ds(base, CHK)], rcs[b], sems[b])
            pltpu.async_copy(ew_hbm.at[pl.ds(base, CHK)], ews[b], sems[b])

        def wait_chunk(ci, b):
            base = cid * EPC + eh * EPH + ci * CHK
            pltpu.make_async_copy(rc_hbm.at[pl.ds(base, CHK)], rcs[b],
                                  sems[b]).wait()
            pltpu.make_async_copy(ew_hbm.at[pl.ds(base, CHK)], ews[b],
                                  sems[b]).wait()

        def process(b):
            @plsc.parallel_loop(0, CHK // 16, unroll=8)
            def ibody(i):
                sl = pl.ds(i * 16, 16)
                e16 = rcs[b][sl]
                r16 = e16 & 16383
                c16 = e16 >> 14
                w16 = ews[b][sl]
                for k in range(FB):
                    v = plsc.load_gather(hcols[k], [r16]) * w16
                    plsc.addupdate_scatter(accs[k], [c16], v)

        start_chunk(0, 0)

        def body(ck, _):
            c0, c1 = 2 * ck, 2 * ck + 1
            wait_chunk(c0, 0)
            start_chunk(c1, 1)
            process(0)
            wait_chunk(c1, 1)

            @pl.when(ck < NCHKH // 2 - 1)
            def _n():
                start_chunk(c0 + 2, 0)

            process(1)
            return _

        lax.fori_loop(0, NCHKH // 2, body, None)
        for k in range(FB):
            pltpu.async_copy(accs[k], out_hbm.at[cid, eh, fg * FB + k],
                             sems[0])
        for k in range(FB):
            pltpu.make_async_copy(accs[k], out_hbm.at[cid, eh, fg * FB + k],
                                  sems[0]).wait()

    return _sc_agg


_sc_agg1 = _make_sc_agg(H1)
_sc_agg2 = _make_sc_agg(H2)


def _tc_h1_body(seq_ref, w1_ref, degp_ref, out_ref):
    dinv = lax.rsqrt(degp_ref[0] + degp_ref[1])
    out_ref[...] = lax.dot_general(
        w1_ref[...], seq_ref[...], (((1,), (1,)), ((), ())),
        preferred_element_type=jnp.float32) * dinv[None, :]


def _tc_mid_body(p_ref, degp_ref, b1_ref, w2_ref, out_ref):
    dinv = lax.rsqrt(degp_ref[0] + degp_ref[1])
    psum = (p_ref[0] + p_ref[1]) + (p_ref[2] + p_ref[3])
    x1 = jax.nn.relu(psum * dinv[None, :] + b1_ref[...])
    out_ref[...] = lax.dot_general(
        w2_ref[...], x1, (((1,), (0,)), ((), ())),
        preferred_element_type=jnp.float32) * dinv[None, :]


def _tc_fin_body(q_ref, degp_ref, b2_ref, wr_ref, wz_ref, wn_ref,
                 br_ref, bz_ref, bni_ref, bnh_ref, wf_ref, bf_ref, out_ref):
    dinv = lax.rsqrt(degp_ref[0] + degp_ref[1])
    qsum = (q_ref[0] + q_ref[1]) + (q_ref[2] + q_ref[3])
    x2 = jax.nn.relu(qsum * dinv[None, :] + b2_ref[...])

    def mm(w, x):
        return lax.dot_general(w[...], x, (((1,), (0,)), ((), ())),
                               preferred_element_type=jnp.float32)

    r = jax.nn.sigmoid(mm(wr_ref, x2) + br_ref[...])
    z = jax.nn.sigmoid(mm(wz_ref, x2) + bz_ref[...])
    n = jnp.tanh(mm(wn_ref, x2) + bni_ref[...] + r * bnh_ref[...])
    hn = (1.0 - z) * n
    out_ref[...] = mm(wf_ref, hn) + bf_ref[0]


def _tc_call(body, out_shape, *args, in_specs=None):
    kw = {} if in_specs is None else {"in_specs": in_specs}
    return pl.pallas_call(body, out_shape=out_shape, **kw)(*args)


def kernel(sequences, edge_weight, W1, b1, W2, b2, W_ih, W_hh, b_ih, b_hh,
           Wf, bf, state_indices, edge_index):
    f32, i32 = jnp.float32, jnp.int32

    seq_pad = jnp.zeros((NP, WINDOW), f32).at[:N_STATES].set(sequences)
    nfill = EPAD - N_EDGES - NP
    loops = jnp.arange(NP, dtype=i32)
    fill_i = jnp.zeros((nfill,), i32)
    row = jnp.concatenate([edge_index[0], loops, fill_i])
    col = jnp.concatenate([edge_index[1], loops, fill_i])
    ew = jnp.concatenate([edge_weight, jnp.ones((NP,), f32),
                          jnp.zeros((nfill,), f32)])
    rc = row | (col << 14)
    col2 = col.reshape(NW, NBLK, EB)
    ew2 = ew.reshape(NW, NBLK, EB)

    degp = _sc_deg(col2, ew2)
    h1t = _tc_call(_tc_h1_body, jax.ShapeDtypeStruct((H1, NP), f32),
                   seq_pad, W1, degp)
    p1 = _sc_agg1(h1t, rc, ew).reshape(4, H1, NP)
    h2t = _tc_call(_tc_mid_body, jax.ShapeDtypeStruct((H2, NP), f32),
                   p1, degp, b1.reshape(H1, 1), W2)
    p2 = _sc_agg2(h2t, rc, ew).reshape(4, H2, NP)

    Wr, Wz, Wn = W_ih[:GRU_H], W_ih[GRU_H:2 * GRU_H], W_ih[2 * GRU_H:]
    br = (b_ih[:GRU_H] + b_hh[:GRU_H]).reshape(GRU_H, 1)
    bz = (b_ih[GRU_H:2 * GRU_H] + b_hh[GRU_H:2 * GRU_H]).reshape(GRU_H, 1)
    bni = b_ih[2 * GRU_H:].reshape(GRU_H, 1)
    bnh = b_hh[2 * GRU_H:].reshape(GRU_H, 1)
    Wf8 = jnp.zeros((8, GRU_H), f32).at[:1].set(Wf)
    out = _tc_call(_tc_fin_body, jax.ShapeDtypeStruct((8, NP), f32),
                   p2, degp, b2.reshape(H2, 1), Wr, Wz, Wn, br, bz, bni, bnh,
                   Wf8, bf,
                   in_specs=[pl.BlockSpec()] * 11
                   + [pl.BlockSpec(memory_space=pltpu.SMEM)])
    return out[0, :N_STATES].reshape(N_STATES, 1)
